# baseline (speedup 1.0000x reference)
import functools

import jax
import jax.numpy as jnp
from jax import lax
from jax.experimental import pallas as pl
from jax.experimental.pallas import tpu as pltpu
from jax.experimental.pallas import tpu_sc as plsc

N_NODES = 3143
N_PAD = 3200
T_STEPS = 48
FEAT = 10
FEAT_PAD = 16
EMBED = 64
HID = 128
N_EDGES = 50288

EDGES_PER_TILE = 3200
E_PAD = EDGES_PER_TILE * 16
SLAB = 400
N_SLABS = N_PAD // SLAB
SLABS_PER_CORE = N_SLABS // 2
ROWS_PER_TILE = SLAB // 16
STRIPE = ROWS_PER_TILE * N_PAD
N_EDGE_GROUPS = EDGES_PER_TILE // 16
N_GROUPS = 208
STAGE_ROWS = N_GROUPS // 8

M_TILE = 320
M_BLOCKS = N_PAD // M_TILE
N_COLS = T_STEPS * EMBED
N_TILE = 512
N_BLOCKS = N_COLS // N_TILE


def _build_adjacency(row, col, ew, zeros_stripe):
    mesh = plsc.VectorSubcoreMesh(core_axis_name="c", subcore_axis_name="s")

    @functools.partial(
        pl.kernel,
        mesh=mesh,
        out_type=jax.ShapeDtypeStruct((N_PAD * N_PAD,), jnp.float32),
        scratch_types=[
            pltpu.VMEM((EDGES_PER_TILE,), jnp.int32),
            pltpu.VMEM((EDGES_PER_TILE,), jnp.int32),
            pltpu.VMEM((EDGES_PER_TILE,), jnp.float32),
            pltpu.VMEM((N_GROUPS * 16,), jnp.int32),
            pltpu.VMEM((N_GROUPS * 16,), jnp.float32),
            pltpu.VMEM_SHARED((SLAB * N_PAD,), jnp.float32),
        ],
    )
    def adj_kernel(row_hbm, col_hbm, ew_hbm, zeros_hbm, a_hbm,
                   row_v, col_v, ew_v, idx_st, val_st, acc_sh):
        cid = lax.axis_index("c")
        sid = lax.axis_index("s")
        ebase = sid * EDGES_PER_TILE
        pltpu.sync_copy(row_hbm.at[pl.ds(ebase, EDGES_PER_TILE)], row_v)
        pltpu.sync_copy(col_hbm.at[pl.ds(ebase, EDGES_PER_TILE)], col_v)
        pltpu.sync_copy(ew_hbm.at[pl.ds(ebase, EDGES_PER_TILE)], ew_v)
        zero16f = jnp.zeros((16,), jnp.float32)
        lanes = lax.iota(jnp.int32, 16)
        for g in range(N_EDGE_GROUPS + 2, N_GROUPS):
            idx_st[pl.ds(g * 16, 16)] = (
                sid * (EDGES_PER_TILE + 32) + g * 16 + lanes)
            val_st[pl.ds(g * 16, 16)] = zero16f

        def do_slab(k, carry):
            lo = (cid * SLABS_PER_CORE + k) * SLAB
            pltpu.sync_copy(zeros_hbm, acc_sh.at[pl.ds(sid * STRIPE, STRIPE)])
            plsc.subcore_barrier()

            def grp(g, c2):
                cg = col_v[pl.ds(g * 16, 16)]
                rg = row_v[pl.ds(g * 16, 16)]
                wg = ew_v[pl.ds(g * 16, 16)]
                m = (cg >= lo) & (cg < lo + SLAB)
                dummy = sid * (EDGES_PER_TILE + 32) + g * 16 + lanes
                fidx = jnp.where(m, (cg - lo) * N_PAD + rg, dummy)
                fval = jnp.where(m, wg, 0.0)
                idx_st[pl.ds(g * 16, 16)] = fidx
                val_st[pl.ds(g * 16, 16)] = fval
                return c2

            lax.fori_loop(0, N_EDGE_GROUPS, grp, 0)
            for j in range(2):
                ii = j * 16 + lanes
                local_c = sid * ROWS_PER_TILE + ii
                m = (ii < ROWS_PER_TILE) & (lo + local_c < N_NODES)
                dummy = sid * (EDGES_PER_TILE + 32) + (N_EDGE_GROUPS + j) * 16 + lanes
                fidx = jnp.where(m, local_c * N_PAD + (lo + local_c), dummy)
                fval = jnp.where(m, jnp.float32(1.0), jnp.float32(0.0))
                g = N_EDGE_GROUPS + j
                idx_st[pl.ds(g * 16, 16)] = fidx
                val_st[pl.ds(g * 16, 16)] = fval
            pltpu.sync_copy(val_st, acc_sh.at[idx_st], add=True)
            plsc.subcore_barrier()
            pltpu.sync_copy(acc_sh.at[pl.ds(sid * STRIPE, STRIPE)],
                            a_hbm.at[pl.ds(lo * N_PAD + sid * STRIPE, STRIPE)])
            return carry

        lax.fori_loop(0, SLABS_PER_CORE, do_slab, 0)

    return adj_kernel(row, col, ew, zeros_stripe)


def _rowsum_rsqrt(a2d):
    def body(a_ref, d_ref):
        s = jnp.sum(a_ref[...], axis=1, keepdims=True)
        d_ref[...] = jnp.where(s > 0, lax.rsqrt(s), 0.0)

    return pl.pallas_call(
        body,
        grid=(M_BLOCKS,),
        in_specs=[pl.BlockSpec((M_TILE, N_PAD), lambda i: (i, 0))],
        out_specs=pl.BlockSpec((M_TILE, 1), lambda i: (i, 0)),
        out_shape=jax.ShapeDtypeStruct((N_PAD, 1), jnp.float32),
    )(a2d)


def _colscale_cast(a2d, d_row):
    def body(a_ref, d_ref, o_ref):
        o_ref[...] = (a_ref[...] * d_ref[...]).astype(jnp.bfloat16)

    return pl.pallas_call(
        body,
        grid=(M_BLOCKS,),
        in_specs=[
            pl.BlockSpec((M_TILE, N_PAD), lambda i: (i, 0)),
            pl.BlockSpec((1, N_PAD), lambda i: (0, 0)),
        ],
        out_specs=pl.BlockSpec((M_TILE, N_PAD), lambda i: (i, 0)),
        out_shape=jax.ShapeDtypeStruct((N_PAD, N_PAD), jnp.bfloat16),
    )(a2d, d_row)


def _input_proj(x_lanes, w1bd):
    def body(x_ref, w_ref, o_ref):
        o_ref[...] = jnp.dot(x_ref[...], w_ref[...],
                             preferred_element_type=jnp.float32).astype(jnp.bfloat16)

    return pl.pallas_call(
        body,
        grid=(N_BLOCKS,),
        in_specs=[
            pl.BlockSpec((N_PAD, T_STEPS * FEAT), lambda j: (0, 0)),
            pl.BlockSpec((T_STEPS * FEAT, N_TILE), lambda j: (0, j)),
        ],
        out_specs=pl.BlockSpec((N_PAD, N_TILE), lambda j: (0, j)),
        out_shape=jax.ShapeDtypeStruct((N_PAD, N_COLS), jnp.bfloat16),
    )(x_lanes, w1bd)


def _gcn_layer1(a_cs, u, d_col, b1rep, w2bd):
    def body(a_ref, u_ref, d_ref, b_ref, w_ref, o_ref):
        acc = jnp.dot(a_ref[...], u_ref[...], preferred_element_type=jnp.float32)
        h = jnp.maximum(acc * d_ref[...] + b_ref[...], 0.0).astype(jnp.bfloat16)
        o_ref[...] = jnp.dot(h, w_ref[...],
                             preferred_element_type=jnp.float32).astype(jnp.bfloat16)

    return pl.pallas_call(
        body,
        grid=(N_BLOCKS,),
        in_specs=[
            pl.BlockSpec((N_PAD, N_PAD), lambda j: (0, 0)),
            pl.BlockSpec((N_PAD, N_TILE), lambda j: (0, j)),
            pl.BlockSpec((N_PAD, 1), lambda j: (0, 0)),
            pl.BlockSpec((1, N_TILE), lambda j: (0, 0)),
            pl.BlockSpec((N_TILE, N_TILE), lambda j: (0, 0)),
        ],
        out_specs=pl.BlockSpec((N_PAD, N_TILE), lambda j: (0, j)),
        out_shape=jax.ShapeDtypeStruct((N_PAD, N_COLS), jnp.bfloat16),
    )(a_cs, u, d_col, b1rep, w2bd)


def _gcn_layer2(a_cs, p, d_col, b2rep, cbp):
    def body(a_ref, p_ref, d_ref, b_ref, cb_ref, o_ref):
        acc = jnp.dot(a_ref[...], p_ref[...], preferred_element_type=jnp.float32)
        cb = jnp.concatenate([cb_ref[...]] * (N_TILE // EMBED), axis=1)
        o_ref[...] = (acc * d_ref[...] + b_ref[...] + cb).astype(jnp.bfloat16)

    return pl.pallas_call(
        body,
        grid=(N_BLOCKS,),
        in_specs=[
            pl.BlockSpec((N_PAD, N_PAD), lambda j: (0, 0)),
            pl.BlockSpec((N_PAD, N_TILE), lambda j: (0, j)),
            pl.BlockSpec((N_PAD, 1), lambda j: (0, 0)),
            pl.BlockSpec((1, N_TILE), lambda j: (0, 0)),
            pl.BlockSpec((N_PAD, EMBED), lambda j: (0, 0)),
        ],
        out_specs=pl.BlockSpec((N_PAD, N_TILE), lambda j: (0, j)),
        out_shape=jax.ShapeDtypeStruct((N_PAD, N_COLS), jnp.bfloat16),
    )(a_cs, p, d_col, b2rep, cbp)


def _lstm_head(e, wcat_t, bsum, w1m_t, b1m, w2m_t, b2m):
    def body(e_ref, w_ref, b_ref, w1m_ref, b1m_ref, w2m_ref,
             b2m_ref, o_ref, xh_sc, h_sc, c_sc):
        g = pl.program_id(0)

        @pl.when(g == 0)
        def _init():
            h_sc[...] = jnp.zeros_like(h_sc)
            c_sc[...] = jnp.zeros_like(c_sc)

        for sub in range(8):
            xh_sc[:, :EMBED] = e_ref[:, sub * EMBED:(sub + 1) * EMBED]
            xh_sc[:, EMBED:] = h_sc[...]
            gates = jnp.dot(xh_sc[...], w_ref[...],
                            preferred_element_type=jnp.float32) + b_ref[...]
            ig = 0.5 + 0.5 * jnp.tanh(0.5 * gates[:, 0 * HID:1 * HID])
            fg = 0.5 + 0.5 * jnp.tanh(0.5 * gates[:, 1 * HID:2 * HID])
            gg = jnp.tanh(gates[:, 2 * HID:3 * HID])
            og = 0.5 + 0.5 * jnp.tanh(0.5 * gates[:, 3 * HID:4 * HID])
            c = fg * c_sc[...] + ig * gg
            h_sc[...] = (og * jnp.tanh(c)).astype(jnp.bfloat16)
            c_sc[...] = c

        @pl.when(g == pl.num_programs(0) - 1)
        def _head():
            hh = jnp.maximum(
                jnp.dot(h_sc[...], w1m_ref[...],
                        preferred_element_type=jnp.float32) + b1m_ref[...], 0.0)
            o_ref[...] = jnp.dot(hh.astype(jnp.bfloat16), w2m_ref[...],
                                 preferred_element_type=jnp.float32) + b2m_ref[...]

    return pl.pallas_call(
        body,
        grid=(T_STEPS // 8,),
        in_specs=[
            pl.BlockSpec((N_PAD, 8 * EMBED), lambda g: (0, g)),
            pl.BlockSpec((EMBED + HID, 4 * HID), lambda g: (0, 0)),
            pl.BlockSpec((1, 4 * HID), lambda g: (0, 0)),
            pl.BlockSpec((HID, EMBED), lambda g: (0, 0)),
            pl.BlockSpec((1, EMBED), lambda g: (0, 0)),
            pl.BlockSpec((EMBED, 1), lambda g: (0, 0)),
            pl.BlockSpec((1, 1), lambda g: (0, 0)),
        ],
        out_specs=pl.BlockSpec((N_PAD, 1), lambda g: (0, 0)),
        out_shape=jax.ShapeDtypeStruct((N_PAD, 1), jnp.float32),
        scratch_shapes=[
            pltpu.VMEM((N_PAD, EMBED + HID), jnp.bfloat16),
            pltpu.VMEM((N_PAD, HID), jnp.bfloat16),
            pltpu.VMEM((N_PAD, HID), jnp.float32),
        ],
    )(e, wcat_t, bsum, w1m_t, b1m, w2m_t, b2m)


def kernel(weather_seq, edge_index, edge_weight, gcn1_W, gcn1_b, gcn2_W,
           gcn2_b, county_bias, lstm_W_ih, lstm_W_hh, lstm_b_ih, lstm_b_hh,
           mlp_W1, mlp_b1, mlp_W2, mlp_b2):
    f32 = jnp.float32
    ei = edge_index.astype(jnp.int32)
    row = jnp.pad(ei[0], (0, E_PAD - N_EDGES))
    col = jnp.pad(ei[1], (0, E_PAD - N_EDGES))
    ew = jnp.pad(edge_weight.astype(f32), (0, E_PAD - N_EDGES))
    zeros_stripe = jnp.zeros((STRIPE,), f32)

    a2d = _build_adjacency(row, col, ew, zeros_stripe).reshape(N_PAD, N_PAD)
    d_col = _rowsum_rsqrt(a2d)
    a_cs = _colscale_cast(a2d, d_col.reshape(1, N_PAD))

    xb = weather_seq.astype(jnp.bfloat16)
    xp = jnp.pad(xb, ((0, 0), (0, N_PAD - N_NODES), (0, 0)))
    x_lanes = jnp.transpose(xp, (1, 0, 2)).reshape(N_PAD, T_STEPS * FEAT)
    eye8 = jnp.eye(8, dtype=f32)
    w1bd = jnp.kron(jnp.eye(T_STEPS, dtype=f32),
                    gcn1_W.T).astype(jnp.bfloat16)
    w2bd = jnp.kron(eye8, gcn2_W.T).astype(jnp.bfloat16)
    b1rep = jnp.tile(gcn1_b, 8)[None]
    b2rep = jnp.tile(gcn2_b, 8)[None]
    cbp = jnp.pad(county_bias, ((0, N_PAD - N_NODES), (0, 0)))

    u = _input_proj(x_lanes, w1bd)
    p = _gcn_layer1(a_cs, u, d_col, b1rep, w2bd)
    e = _gcn_layer2(a_cs, p, d_col, b2rep, cbp)

    bf = jnp.bfloat16
    wcat = jnp.concatenate([lstm_W_ih.T, lstm_W_hh.T], axis=0).astype(bf)
    out = _lstm_head(e, wcat, (lstm_b_ih + lstm_b_hh)[None],
                     mlp_W1.T.astype(bf), mlp_b1[None],
                     mlp_W2.T.astype(bf), mlp_b2[None])
    return out[:N_NODES, 0]


# speedup vs baseline: 1.0091x; 1.0091x over previous
import functools

import jax
import jax.numpy as jnp
from jax import lax
from jax.experimental import pallas as pl
from jax.experimental.pallas import tpu as pltpu
from jax.experimental.pallas import tpu_sc as plsc

N_NODES = 3143
N_PAD = 3200
T_STEPS = 48
FEAT = 10
FEAT_PAD = 16
EMBED = 64
HID = 128
N_EDGES = 50288

EDGES_PER_TILE = 3200
E_PAD = EDGES_PER_TILE * 16
SLAB = 400
N_SLABS = N_PAD // SLAB
SLABS_PER_CORE = N_SLABS // 2
ROWS_PER_TILE = SLAB // 16
STRIPE = ROWS_PER_TILE * N_PAD
N_EDGE_GROUPS = EDGES_PER_TILE // 16
N_GROUPS = 208
STAGE_ROWS = N_GROUPS // 8

M_TILE = 320
M_BLOCKS = N_PAD // M_TILE
N_COLS = T_STEPS * EMBED
N_TILE = 512
N_BLOCKS = N_COLS // N_TILE


def _build_adjacency(row, col, ew, zeros_stripe):
    mesh = plsc.VectorSubcoreMesh(core_axis_name="c", subcore_axis_name="s")

    @functools.partial(
        pl.kernel,
        mesh=mesh,
        out_type=jax.ShapeDtypeStruct((N_PAD * N_PAD,), jnp.float32),
        scratch_types=[
            pltpu.VMEM((EDGES_PER_TILE,), jnp.int32),
            pltpu.VMEM((EDGES_PER_TILE,), jnp.int32),
            pltpu.VMEM((EDGES_PER_TILE,), jnp.float32),
            pltpu.VMEM((N_GROUPS * 16,), jnp.int32),
            pltpu.VMEM((N_GROUPS * 16,), jnp.float32),
            pltpu.VMEM_SHARED((SLAB * N_PAD,), jnp.float32),
        ],
    )
    def adj_kernel(row_hbm, col_hbm, ew_hbm, zeros_hbm, a_hbm,
                   row_v, col_v, ew_v, idx_st, val_st, acc_sh):
        cid = lax.axis_index("c")
        sid = lax.axis_index("s")
        ebase = sid * EDGES_PER_TILE
        pltpu.sync_copy(row_hbm.at[pl.ds(ebase, EDGES_PER_TILE)], row_v)
        pltpu.sync_copy(col_hbm.at[pl.ds(ebase, EDGES_PER_TILE)], col_v)
        pltpu.sync_copy(ew_hbm.at[pl.ds(ebase, EDGES_PER_TILE)], ew_v)
        zero16f = jnp.zeros((16,), jnp.float32)
        lanes = lax.iota(jnp.int32, 16)
        for g in range(N_EDGE_GROUPS + 2, N_GROUPS):
            idx_st[pl.ds(g * 16, 16)] = (
                sid * (EDGES_PER_TILE + 32) + g * 16 + lanes)
            val_st[pl.ds(g * 16, 16)] = zero16f

        def do_slab(k, carry):
            lo = (cid * SLABS_PER_CORE + k) * SLAB
            pltpu.sync_copy(zeros_hbm, acc_sh.at[pl.ds(sid * STRIPE, STRIPE)])
            plsc.subcore_barrier()

            def grp(g, c2):
                cg = col_v[pl.ds(g * 16, 16)]
                rg = row_v[pl.ds(g * 16, 16)]
                wg = ew_v[pl.ds(g * 16, 16)]
                m = (cg >= lo) & (cg < lo + SLAB)
                dummy = sid * (EDGES_PER_TILE + 32) + g * 16 + lanes
                fidx = jnp.where(m, (cg - lo) * N_PAD + rg, dummy)
                fval = jnp.where(m, wg, 0.0)
                idx_st[pl.ds(g * 16, 16)] = fidx
                val_st[pl.ds(g * 16, 16)] = fval
                return c2

            lax.fori_loop(0, N_EDGE_GROUPS, grp, 0)
            for j in range(2):
                ii = j * 16 + lanes
                local_c = sid * ROWS_PER_TILE + ii
                m = (ii < ROWS_PER_TILE) & (lo + local_c < N_NODES)
                dummy = sid * (EDGES_PER_TILE + 32) + (N_EDGE_GROUPS + j) * 16 + lanes
                fidx = jnp.where(m, local_c * N_PAD + (lo + local_c), dummy)
                fval = jnp.where(m, jnp.float32(1.0), jnp.float32(0.0))
                g = N_EDGE_GROUPS + j
                idx_st[pl.ds(g * 16, 16)] = fidx
                val_st[pl.ds(g * 16, 16)] = fval
            pltpu.sync_copy(val_st, acc_sh.at[idx_st], add=True)
            plsc.subcore_barrier()
            pltpu.sync_copy(acc_sh.at[pl.ds(sid * STRIPE, STRIPE)],
                            a_hbm.at[pl.ds(lo * N_PAD + sid * STRIPE, STRIPE)])
            return carry

        lax.fori_loop(0, SLABS_PER_CORE, do_slab, 0)

    return adj_kernel(row, col, ew, zeros_stripe)


def _rowsum_rsqrt(a2d):
    def body(a_ref, d_ref):
        s = jnp.sum(a_ref[...], axis=1, keepdims=True)
        d_ref[...] = jnp.where(s > 0, lax.rsqrt(s), 0.0)

    return pl.pallas_call(
        body,
        grid=(M_BLOCKS,),
        in_specs=[pl.BlockSpec((M_TILE, N_PAD), lambda i: (i, 0))],
        out_specs=pl.BlockSpec((M_TILE, 1), lambda i: (i, 0)),
        out_shape=jax.ShapeDtypeStruct((N_PAD, 1), jnp.float32),
    )(a2d)


def _colscale_cast(a2d, d_row):
    def body(a_ref, d_ref, o_ref):
        o_ref[...] = (a_ref[...] * d_ref[...]).astype(jnp.bfloat16)

    return pl.pallas_call(
        body,
        grid=(M_BLOCKS,),
        in_specs=[
            pl.BlockSpec((M_TILE, N_PAD), lambda i: (i, 0)),
            pl.BlockSpec((1, N_PAD), lambda i: (0, 0)),
        ],
        out_specs=pl.BlockSpec((M_TILE, N_PAD), lambda i: (i, 0)),
        out_shape=jax.ShapeDtypeStruct((N_PAD, N_PAD), jnp.bfloat16),
    )(a2d, d_row)


def _input_proj(x_lanes, w1bd):
    def body(x_ref, w_ref, o_ref):
        o_ref[...] = jnp.dot(x_ref[...], w_ref[...],
                             preferred_element_type=jnp.float32).astype(jnp.bfloat16)

    return pl.pallas_call(
        body,
        grid=(N_BLOCKS,),
        in_specs=[
            pl.BlockSpec((N_PAD, T_STEPS * FEAT), lambda j: (0, 0)),
            pl.BlockSpec((T_STEPS * FEAT, N_TILE), lambda j: (0, j)),
        ],
        out_specs=pl.BlockSpec((N_PAD, N_TILE), lambda j: (0, j)),
        out_shape=jax.ShapeDtypeStruct((N_PAD, N_COLS), jnp.bfloat16),
    )(x_lanes, w1bd)


def _gcn_layer1(a_cs, u, d_col, b1rep, w2bd):
    def body(a_ref, u_ref, d_ref, b_ref, w_ref, o_ref):
        acc = jnp.dot(a_ref[...], u_ref[...], preferred_element_type=jnp.float32)
        h = jnp.maximum(acc * d_ref[...] + b_ref[...], 0.0).astype(jnp.bfloat16)
        o_ref[...] = jnp.dot(h, w_ref[...],
                             preferred_element_type=jnp.float32).astype(jnp.bfloat16)

    return pl.pallas_call(
        body,
        grid=(N_BLOCKS,),
        in_specs=[
            pl.BlockSpec((N_PAD, N_PAD), lambda j: (0, 0)),
            pl.BlockSpec((N_PAD, N_TILE), lambda j: (0, j)),
            pl.BlockSpec((N_PAD, 1), lambda j: (0, 0)),
            pl.BlockSpec((1, N_TILE), lambda j: (0, 0)),
            pl.BlockSpec((N_TILE, N_TILE), lambda j: (0, 0)),
        ],
        out_specs=pl.BlockSpec((N_PAD, N_TILE), lambda j: (0, j)),
        out_shape=jax.ShapeDtypeStruct((N_PAD, N_COLS), jnp.bfloat16),
    )(a_cs, u, d_col, b1rep, w2bd)


def _gcn_layer2(a_cs, p, d_col, b2rep, cbp):
    def body(a_ref, p_ref, d_ref, b_ref, cb_ref, o_ref):
        acc = jnp.dot(a_ref[...], p_ref[...], preferred_element_type=jnp.float32)
        cb = jnp.concatenate([cb_ref[...]] * (N_TILE // EMBED), axis=1)
        o_ref[...] = (acc * d_ref[...] + b_ref[...] + cb).astype(jnp.bfloat16)

    return pl.pallas_call(
        body,
        grid=(N_BLOCKS,),
        in_specs=[
            pl.BlockSpec((N_PAD, N_PAD), lambda j: (0, 0)),
            pl.BlockSpec((N_PAD, N_TILE), lambda j: (0, j)),
            pl.BlockSpec((N_PAD, 1), lambda j: (0, 0)),
            pl.BlockSpec((1, N_TILE), lambda j: (0, 0)),
            pl.BlockSpec((N_PAD, EMBED), lambda j: (0, 0)),
        ],
        out_specs=pl.BlockSpec((N_PAD, N_TILE), lambda j: (0, j)),
        out_shape=jax.ShapeDtypeStruct((N_PAD, N_COLS), jnp.bfloat16),
    )(a_cs, p, d_col, b2rep, cbp)


def _lstm_head(e, wcat_t, bsum, w1m_t, b1m, w2m_t, b2m):
    def body(e_ref, w_ref, b_ref, w1m_ref, b1m_ref, w2m_ref,
             b2m_ref, o_ref, xh_sc, h_sc, c_sc):
        g = pl.program_id(0)

        @pl.when(g == 0)
        def _init():
            h_sc[...] = jnp.zeros_like(h_sc)
            c_sc[...] = jnp.zeros_like(c_sc)

        for sub in range(4):
            xh_sc[:, :EMBED] = e_ref[:, sub * EMBED:(sub + 1) * EMBED]
            xh_sc[:, EMBED:] = h_sc[...]
            gates = jnp.dot(xh_sc[...], w_ref[...],
                            preferred_element_type=jnp.float32) + b_ref[...]
            ig = 0.5 + 0.5 * jnp.tanh(0.5 * gates[:, 0 * HID:1 * HID])
            fg = 0.5 + 0.5 * jnp.tanh(0.5 * gates[:, 1 * HID:2 * HID])
            gg = jnp.tanh(gates[:, 2 * HID:3 * HID])
            og = 0.5 + 0.5 * jnp.tanh(0.5 * gates[:, 3 * HID:4 * HID])
            c = fg * c_sc[...] + ig * gg
            h_sc[...] = (og * jnp.tanh(c)).astype(jnp.bfloat16)
            c_sc[...] = c

        @pl.when(g == pl.num_programs(0) - 1)
        def _head():
            hh = jnp.maximum(
                jnp.dot(h_sc[...], w1m_ref[...],
                        preferred_element_type=jnp.float32) + b1m_ref[...], 0.0)
            o_ref[...] = jnp.dot(hh.astype(jnp.bfloat16), w2m_ref[...],
                                 preferred_element_type=jnp.float32) + b2m_ref[...]

    return pl.pallas_call(
        body,
        grid=(T_STEPS // 4,),
        in_specs=[
            pl.BlockSpec((N_PAD, 4 * EMBED), lambda g: (0, g)),
            pl.BlockSpec((EMBED + HID, 4 * HID), lambda g: (0, 0)),
            pl.BlockSpec((1, 4 * HID), lambda g: (0, 0)),
            pl.BlockSpec((HID, EMBED), lambda g: (0, 0)),
            pl.BlockSpec((1, EMBED), lambda g: (0, 0)),
            pl.BlockSpec((EMBED, 1), lambda g: (0, 0)),
            pl.BlockSpec((1, 1), lambda g: (0, 0)),
        ],
        out_specs=pl.BlockSpec((N_PAD, 1), lambda g: (0, 0)),
        out_shape=jax.ShapeDtypeStruct((N_PAD, 1), jnp.float32),
        scratch_shapes=[
            pltpu.VMEM((N_PAD, EMBED + HID), jnp.bfloat16),
            pltpu.VMEM((N_PAD, HID), jnp.bfloat16),
            pltpu.VMEM((N_PAD, HID), jnp.float32),
        ],
    )(e, wcat_t, bsum, w1m_t, b1m, w2m_t, b2m)


def kernel(weather_seq, edge_index, edge_weight, gcn1_W, gcn1_b, gcn2_W,
           gcn2_b, county_bias, lstm_W_ih, lstm_W_hh, lstm_b_ih, lstm_b_hh,
           mlp_W1, mlp_b1, mlp_W2, mlp_b2):
    f32 = jnp.float32
    ei = edge_index.astype(jnp.int32)
    row = jnp.pad(ei[0], (0, E_PAD - N_EDGES))
    col = jnp.pad(ei[1], (0, E_PAD - N_EDGES))
    ew = jnp.pad(edge_weight.astype(f32), (0, E_PAD - N_EDGES))
    zeros_stripe = jnp.zeros((STRIPE,), f32)

    a2d = _build_adjacency(row, col, ew, zeros_stripe).reshape(N_PAD, N_PAD)
    d_col = _rowsum_rsqrt(a2d)
    a_cs = _colscale_cast(a2d, d_col.reshape(1, N_PAD))

    xb = weather_seq.astype(jnp.bfloat16)
    xp = jnp.pad(xb, ((0, 0), (0, N_PAD - N_NODES), (0, 0)))
    x_lanes = jnp.transpose(xp, (1, 0, 2)).reshape(N_PAD, T_STEPS * FEAT)
    eye8 = jnp.eye(8, dtype=f32)
    w1bd = jnp.kron(jnp.eye(T_STEPS, dtype=f32),
                    gcn1_W.T).astype(jnp.bfloat16)
    w2bd = jnp.kron(eye8, gcn2_W.T).astype(jnp.bfloat16)
    b1rep = jnp.tile(gcn1_b, 8)[None]
    b2rep = jnp.tile(gcn2_b, 8)[None]
    cbp = jnp.pad(county_bias, ((0, N_PAD - N_NODES), (0, 0)))

    u = _input_proj(x_lanes, w1bd)
    p = _gcn_layer1(a_cs, u, d_col, b1rep, w2bd)
    e = _gcn_layer2(a_cs, p, d_col, b2rep, cbp)

    bf = jnp.bfloat16
    wcat = jnp.concatenate([lstm_W_ih.T, lstm_W_hh.T], axis=0).astype(bf)
    out = _lstm_head(e, wcat, (lstm_b_ih + lstm_b_hh)[None],
                     mlp_W1.T.astype(bf), mlp_b1[None],
                     mlp_W2.T.astype(bf), mlp_b2[None])
    return out[:N_NODES, 0]


# speedup vs baseline: 1.0442x; 1.0348x over previous
import functools

import jax
import jax.numpy as jnp
from jax import lax
from jax.experimental import pallas as pl
from jax.experimental.pallas import tpu as pltpu
from jax.experimental.pallas import tpu_sc as plsc

N_NODES = 3143
N_PAD = 3200
T_STEPS = 48
FEAT = 10
FEAT_PAD = 16
EMBED = 64
HID = 128
N_EDGES = 50288

EDGES_PER_TILE = 3200
E_PAD = EDGES_PER_TILE * 16
SLAB = 400
N_SLABS = N_PAD // SLAB
SLABS_PER_CORE = N_SLABS // 2
ROWS_PER_TILE = SLAB // 16
STRIPE = ROWS_PER_TILE * N_PAD
N_EDGE_GROUPS = EDGES_PER_TILE // 16
N_GROUPS = 208
STAGE_ROWS = N_GROUPS // 8

M_TILE = 320
M_BLOCKS = N_PAD // M_TILE
N_COLS = T_STEPS * EMBED
N_TILE = 512
N_BLOCKS = N_COLS // N_TILE


def _build_adjacency(row, col, ew, zeros_stripe):
    mesh = plsc.VectorSubcoreMesh(core_axis_name="c", subcore_axis_name="s")

    @functools.partial(
        pl.kernel,
        mesh=mesh,
        out_type=jax.ShapeDtypeStruct((N_PAD * N_PAD,), jnp.float32),
        scratch_types=[
            pltpu.VMEM((EDGES_PER_TILE,), jnp.int32),
            pltpu.VMEM((EDGES_PER_TILE,), jnp.int32),
            pltpu.VMEM((EDGES_PER_TILE,), jnp.float32),
            pltpu.VMEM((N_GROUPS * 16,), jnp.int32),
            pltpu.VMEM((N_GROUPS * 16,), jnp.float32),
            pltpu.VMEM_SHARED((SLAB * N_PAD,), jnp.float32),
        ],
    )
    def adj_kernel(row_hbm, col_hbm, ew_hbm, zeros_hbm, a_hbm,
                   row_v, col_v, ew_v, idx_st, val_st, acc_sh):
        cid = lax.axis_index("c")
        sid = lax.axis_index("s")
        ebase = sid * EDGES_PER_TILE
        pltpu.sync_copy(row_hbm.at[pl.ds(ebase, EDGES_PER_TILE)], row_v)
        pltpu.sync_copy(col_hbm.at[pl.ds(ebase, EDGES_PER_TILE)], col_v)
        pltpu.sync_copy(ew_hbm.at[pl.ds(ebase, EDGES_PER_TILE)], ew_v)
        zero16f = jnp.zeros((16,), jnp.float32)
        lanes = lax.iota(jnp.int32, 16)
        for g in range(N_EDGE_GROUPS + 2, N_GROUPS):
            idx_st[pl.ds(g * 16, 16)] = (
                sid * (EDGES_PER_TILE + 32) + g * 16 + lanes)
            val_st[pl.ds(g * 16, 16)] = zero16f

        def do_slab(k, carry):
            lo = (cid * SLABS_PER_CORE + k) * SLAB
            pltpu.sync_copy(zeros_hbm, acc_sh.at[pl.ds(sid * STRIPE, STRIPE)])
            plsc.subcore_barrier()

            def grp(g, c2):
                cg = col_v[pl.ds(g * 16, 16)]
                rg = row_v[pl.ds(g * 16, 16)]
                wg = ew_v[pl.ds(g * 16, 16)]
                m = (cg >= lo) & (cg < lo + SLAB)
                dummy = sid * (EDGES_PER_TILE + 32) + g * 16 + lanes
                fidx = jnp.where(m, (cg - lo) * N_PAD + rg, dummy)
                fval = jnp.where(m, wg, 0.0)
                idx_st[pl.ds(g * 16, 16)] = fidx
                val_st[pl.ds(g * 16, 16)] = fval
                return c2

            lax.fori_loop(0, N_EDGE_GROUPS, grp, 0)
            for j in range(2):
                ii = j * 16 + lanes
                local_c = sid * ROWS_PER_TILE + ii
                m = (ii < ROWS_PER_TILE) & (lo + local_c < N_NODES)
                dummy = sid * (EDGES_PER_TILE + 32) + (N_EDGE_GROUPS + j) * 16 + lanes
                fidx = jnp.where(m, local_c * N_PAD + (lo + local_c), dummy)
                fval = jnp.where(m, jnp.float32(1.0), jnp.float32(0.0))
                g = N_EDGE_GROUPS + j
                idx_st[pl.ds(g * 16, 16)] = fidx
                val_st[pl.ds(g * 16, 16)] = fval
            pltpu.sync_copy(val_st, acc_sh.at[idx_st], add=True)
            plsc.subcore_barrier()
            pltpu.sync_copy(acc_sh.at[pl.ds(sid * STRIPE, STRIPE)],
                            a_hbm.at[pl.ds(lo * N_PAD + sid * STRIPE, STRIPE)])
            return carry

        lax.fori_loop(0, SLABS_PER_CORE, do_slab, 0)

    return adj_kernel(row, col, ew, zeros_stripe)


def _norm_rowscale_cast(a2d):
    def body(a_ref, o_ref, d_ref):
        s = jnp.sum(a_ref[...], axis=1, keepdims=True)
        d = jnp.where(s > 0, lax.rsqrt(s), 0.0)
        d_ref[...] = d
        o_ref[...] = (a_ref[...] * d).astype(jnp.bfloat16)

    return pl.pallas_call(
        body,
        grid=(M_BLOCKS,),
        in_specs=[pl.BlockSpec((M_TILE, N_PAD), lambda i: (i, 0))],
        out_specs=[
            pl.BlockSpec((M_TILE, N_PAD), lambda i: (i, 0)),
            pl.BlockSpec((M_TILE, 1), lambda i: (i, 0)),
        ],
        out_shape=[
            jax.ShapeDtypeStruct((N_PAD, N_PAD), jnp.bfloat16),
            jax.ShapeDtypeStruct((N_PAD, 1), jnp.float32),
        ],
    )(a2d)


def _input_proj(x_lanes, w1bd):
    def body(x_ref, w_ref, o_ref):
        o_ref[...] = jnp.dot(x_ref[...], w_ref[...],
                             preferred_element_type=jnp.float32).astype(jnp.bfloat16)

    return pl.pallas_call(
        body,
        grid=(N_BLOCKS,),
        in_specs=[
            pl.BlockSpec((N_PAD, T_STEPS * FEAT), lambda j: (0, 0)),
            pl.BlockSpec((T_STEPS * FEAT, N_TILE), lambda j: (0, j)),
        ],
        out_specs=pl.BlockSpec((N_PAD, N_TILE), lambda j: (0, j)),
        out_shape=jax.ShapeDtypeStruct((N_PAD, N_COLS), jnp.bfloat16),
    )(x_lanes, w1bd)


def _gcn_layer1(a_rs, u, d_col, b1rep, w2bd):
    def body(a_ref, u_ref, d_ref, b_ref, w_ref, o_ref):
        us = (u_ref[...].astype(jnp.float32) * d_ref[...]).astype(jnp.bfloat16)
        acc = jnp.dot(a_ref[...], us, preferred_element_type=jnp.float32)
        h = jnp.maximum(acc + b_ref[...], 0.0).astype(jnp.bfloat16)
        o_ref[...] = jnp.dot(h, w_ref[...],
                             preferred_element_type=jnp.float32).astype(jnp.bfloat16)

    return pl.pallas_call(
        body,
        grid=(N_BLOCKS,),
        in_specs=[
            pl.BlockSpec((N_PAD, N_PAD), lambda j: (0, 0)),
            pl.BlockSpec((N_PAD, N_TILE), lambda j: (0, j)),
            pl.BlockSpec((N_PAD, 1), lambda j: (0, 0)),
            pl.BlockSpec((1, N_TILE), lambda j: (0, 0)),
            pl.BlockSpec((N_TILE, N_TILE), lambda j: (0, 0)),
        ],
        out_specs=pl.BlockSpec((N_PAD, N_TILE), lambda j: (0, j)),
        out_shape=jax.ShapeDtypeStruct((N_PAD, N_COLS), jnp.bfloat16),
    )(a_rs, u, d_col, b1rep, w2bd)


def _gcn_layer2(a_rs, p, d_col, b2rep, cbp):
    def body(a_ref, p_ref, d_ref, b_ref, cb_ref, o_ref):
        ps = (p_ref[...].astype(jnp.float32) * d_ref[...]).astype(jnp.bfloat16)
        acc = jnp.dot(a_ref[...], ps, preferred_element_type=jnp.float32)
        cb = jnp.concatenate([cb_ref[...]] * (N_TILE // EMBED), axis=1)
        o_ref[...] = (acc + b_ref[...] + cb).astype(jnp.bfloat16)

    return pl.pallas_call(
        body,
        grid=(N_BLOCKS,),
        in_specs=[
            pl.BlockSpec((N_PAD, N_PAD), lambda j: (0, 0)),
            pl.BlockSpec((N_PAD, N_TILE), lambda j: (0, j)),
            pl.BlockSpec((N_PAD, 1), lambda j: (0, 0)),
            pl.BlockSpec((1, N_TILE), lambda j: (0, 0)),
            pl.BlockSpec((N_PAD, EMBED), lambda j: (0, 0)),
        ],
        out_specs=pl.BlockSpec((N_PAD, N_TILE), lambda j: (0, j)),
        out_shape=jax.ShapeDtypeStruct((N_PAD, N_COLS), jnp.bfloat16),
    )(a_rs, p, d_col, b2rep, cbp)


def _lstm_head(e, wcat_t, bsum, w1m_t, b1m, w2m_t, b2m):
    def body(e_ref, w_ref, b_ref, w1m_ref, b1m_ref, w2m_ref,
             b2m_ref, o_ref, xh_sc, h_sc, c_sc):
        g = pl.program_id(0)

        @pl.when(g == 0)
        def _init():
            h_sc[...] = jnp.zeros_like(h_sc)
            c_sc[...] = jnp.zeros_like(c_sc)

        for sub in range(4):
            xh_sc[:, :EMBED] = e_ref[:, sub * EMBED:(sub + 1) * EMBED]
            xh_sc[:, EMBED:] = h_sc[...]
            gates = jnp.dot(xh_sc[...], w_ref[...],
                            preferred_element_type=jnp.float32) + b_ref[...]
            ig = 0.5 + 0.5 * jnp.tanh(0.5 * gates[:, 0 * HID:1 * HID])
            fg = 0.5 + 0.5 * jnp.tanh(0.5 * gates[:, 1 * HID:2 * HID])
            gg = jnp.tanh(gates[:, 2 * HID:3 * HID])
            og = 0.5 + 0.5 * jnp.tanh(0.5 * gates[:, 3 * HID:4 * HID])
            c = fg * c_sc[...] + ig * gg
            h_sc[...] = (og * jnp.tanh(c)).astype(jnp.bfloat16)
            c_sc[...] = c

        @pl.when(g == pl.num_programs(0) - 1)
        def _head():
            hh = jnp.maximum(
                jnp.dot(h_sc[...], w1m_ref[...],
                        preferred_element_type=jnp.float32) + b1m_ref[...], 0.0)
            o_ref[...] = jnp.dot(hh.astype(jnp.bfloat16), w2m_ref[...],
                                 preferred_element_type=jnp.float32) + b2m_ref[...]

    return pl.pallas_call(
        body,
        grid=(T_STEPS // 4,),
        in_specs=[
            pl.BlockSpec((N_PAD, 4 * EMBED), lambda g: (0, g)),
            pl.BlockSpec((EMBED + HID, 4 * HID), lambda g: (0, 0)),
            pl.BlockSpec((1, 4 * HID), lambda g: (0, 0)),
            pl.BlockSpec((HID, EMBED), lambda g: (0, 0)),
            pl.BlockSpec((1, EMBED), lambda g: (0, 0)),
            pl.BlockSpec((EMBED, 1), lambda g: (0, 0)),
            pl.BlockSpec((1, 1), lambda g: (0, 0)),
        ],
        out_specs=pl.BlockSpec((N_PAD, 1), lambda g: (0, 0)),
        out_shape=jax.ShapeDtypeStruct((N_PAD, 1), jnp.float32),
        scratch_shapes=[
            pltpu.VMEM((N_PAD, EMBED + HID), jnp.bfloat16),
            pltpu.VMEM((N_PAD, HID), jnp.bfloat16),
            pltpu.VMEM((N_PAD, HID), jnp.float32),
        ],
    )(e, wcat_t, bsum, w1m_t, b1m, w2m_t, b2m)


def kernel(weather_seq, edge_index, edge_weight, gcn1_W, gcn1_b, gcn2_W,
           gcn2_b, county_bias, lstm_W_ih, lstm_W_hh, lstm_b_ih, lstm_b_hh,
           mlp_W1, mlp_b1, mlp_W2, mlp_b2):
    f32 = jnp.float32
    ei = edge_index.astype(jnp.int32)
    row = jnp.pad(ei[0], (0, E_PAD - N_EDGES))
    col = jnp.pad(ei[1], (0, E_PAD - N_EDGES))
    ew = jnp.pad(edge_weight.astype(f32), (0, E_PAD - N_EDGES))
    zeros_stripe = jnp.zeros((STRIPE,), f32)

    a2d = _build_adjacency(row, col, ew, zeros_stripe).reshape(N_PAD, N_PAD)
    a_rs, d_col = _norm_rowscale_cast(a2d)

    xb = weather_seq.astype(jnp.bfloat16)
    xp = jnp.pad(xb, ((0, 0), (0, N_PAD - N_NODES), (0, 0)))
    x_lanes = jnp.transpose(xp, (1, 0, 2)).reshape(N_PAD, T_STEPS * FEAT)
    eye8 = jnp.eye(8, dtype=f32)
    w1bd = jnp.kron(jnp.eye(T_STEPS, dtype=f32),
                    gcn1_W.T).astype(jnp.bfloat16)
    w2bd = jnp.kron(eye8, gcn2_W.T).astype(jnp.bfloat16)
    b1rep = jnp.tile(gcn1_b, 8)[None]
    b2rep = jnp.tile(gcn2_b, 8)[None]
    cbp = jnp.pad(county_bias, ((0, N_PAD - N_NODES), (0, 0)))

    u = _input_proj(x_lanes, w1bd)
    p = _gcn_layer1(a_rs, u, d_col, b1rep, w2bd)
    e = _gcn_layer2(a_rs, p, d_col, b2rep, cbp)

    bf = jnp.bfloat16
    wcat = jnp.concatenate([lstm_W_ih.T, lstm_W_hh.T], axis=0).astype(bf)
    out = _lstm_head(e, wcat, (lstm_b_ih + lstm_b_hh)[None],
                     mlp_W1.T.astype(bf), mlp_b1[None],
                     mlp_W2.T.astype(bf), mlp_b2[None])
    return out[:N_NODES, 0]
